# baseline (speedup 1.0000x reference)
import functools
import math

import jax
import jax.numpy as jnp
from jax import lax
from jax.experimental import pallas as pl
from jax.experimental.pallas import tpu as pltpu
from jax.experimental.pallas import tpu_sc as plsc

B = 1024
L = 200
D = 64
N_POS = 200

_NC = 2
_NS = 16
_LANES = 16
_NW = _NC * _NS
_TOK = B * L
_PER_W = _TOK // _NW
_CHUNK = 100
_NCHUNK = _PER_W // _CHUNK
_SCALE = math.sqrt(float(D))


def _pos_body(out_ref):
    pos = lax.broadcasted_iota(jnp.int32, (L, D), 0).astype(jnp.float32)
    d = lax.broadcasted_iota(jnp.int32, (L, D), 1)
    f = (d // 2).astype(jnp.float32)
    freq = jnp.exp(f * (-2.0 / D) * math.log(float(N_POS)))
    ang = pos * freq
    out_ref[...] = jnp.where(d % 2 == 0, jnp.sin(ang), jnp.cos(ang))


def _pos_enc():
    return pl.pallas_call(
        _pos_body,
        out_shape=jax.ShapeDtypeStruct((L, D), jnp.float32),
    )()


def _sc_body(idx_flat, idx2d, pos, table, out, mask_out,
             idxf_v, idx2d_v, pos_v, rows_v, mask_v, sem):
    wid = lax.axis_index("s") * _NC + lax.axis_index("c")
    base = wid * _PER_W

    pltpu.sync_copy(pos, pos_v)
    pltpu.sync_copy(idx_flat.at[pl.ds(base, _PER_W)], idxf_v)
    pltpu.sync_copy(idx2d.at[wid], idx2d_v)

    def mask_body(k, carry):
        sl = pl.ds(k * _LANES, _LANES)
        mask_v[sl] = jnp.minimum(idxf_v[sl], 1)
        return carry

    lax.fori_loop(0, _PER_W // _LANES, mask_body, 0)
    pltpu.sync_copy(mask_v, mask_out.at[pl.ds(base, _PER_W)])

    def chunk_body(g, carry):
        c0 = pltpu.async_copy(
            table.at[idx2d_v.at[2 * g]], rows_v.at[pl.ds(0, _CHUNK)], sem)
        c1 = pltpu.async_copy(
            table.at[idx2d_v.at[2 * g + 1]], rows_v.at[pl.ds(_CHUNK, _CHUNK)],
            sem)
        c0.wait()
        c1.wait()

        def row_body(r, c2):
            for c in range(D // _LANES):
                sl = pl.ds(c * _LANES, _LANES)
                rows_v[r, sl] = rows_v[r, sl] * _SCALE + pos_v[r, sl]
            return c2

        lax.fori_loop(0, L, row_body, 0)
        pltpu.sync_copy(rows_v, out.at[pl.ds(base + g * L, L)])
        return carry

    lax.fori_loop(0, _PER_W // L, chunk_body, 0)


@functools.partial(jax.jit, static_argnames=())
def _run(idx_flat, idx2d, pos, table):
    mesh = plsc.VectorSubcoreMesh(core_axis_name="c", subcore_axis_name="s")
    f = pl.kernel(
        _sc_body,
        out_type=(
            jax.ShapeDtypeStruct((_TOK, D), jnp.float32),
            jax.ShapeDtypeStruct((_TOK,), jnp.int32),
        ),
        mesh=mesh,
        scratch_types=(
            pltpu.VMEM((_PER_W,), jnp.int32),
            pltpu.VMEM((_NCHUNK, _CHUNK), jnp.int32),
            pltpu.VMEM((L, D), jnp.float32),
            pltpu.VMEM((L, D), jnp.float32),
            pltpu.VMEM((_PER_W,), jnp.int32),
            pltpu.SemaphoreType.DMA,
        ),
        compiler_params=pltpu.CompilerParams(use_tc_tiling_on_sc=False),
    )
    return f(idx_flat, idx2d, pos, table)


def kernel(indices, table):
    idx_flat = indices.reshape(-1).astype(jnp.int32)
    idx2d = idx_flat.reshape(_NW, _NCHUNK, _CHUNK)
    pos = _pos_enc()
    out_flat, mask_i32 = _run(idx_flat, idx2d, pos, table)
    x = out_flat.reshape(B, L, D)
    mask = mask_i32.reshape(B, L).astype(bool)
    return (x, mask)


# speedup vs baseline: 1.0746x; 1.0746x over previous
import functools
import math

import jax
import jax.numpy as jnp
from jax import lax
from jax.experimental import pallas as pl
from jax.experimental.pallas import tpu as pltpu
from jax.experimental.pallas import tpu_sc as plsc

B = 1024
L = 200
D = 64
N_POS = 200

_NC = 2
_NS = 16
_LANES = 16
_NW = _NC * _NS
_TOK = B * L
_PER_W = _TOK // _NW
_CHUNK = 100
_SEQ_W = _PER_W // L
_NB = 4
_SCALE = math.sqrt(float(D))


def _pos_body(out_ref):
    pos = lax.broadcasted_iota(jnp.int32, (L, D), 0).astype(jnp.float32)
    d = lax.broadcasted_iota(jnp.int32, (L, D), 1)
    f = (d // 2).astype(jnp.float32)
    freq = jnp.exp(f * (-2.0 / D) * math.log(float(N_POS)))
    ang = pos * freq
    out_ref[...] = jnp.where(d % 2 == 0, jnp.sin(ang), jnp.cos(ang))


def _pos_enc():
    return pl.pallas_call(
        _pos_body,
        out_shape=jax.ShapeDtypeStruct((L, D), jnp.float32),
    )()


def _sc_body(idx_flat, idx2d, pos, table, out, mask_out,
             idxf_v, idx2d_v, pos_v, r0, r1, r2, r3, mask_v,
             g0, g1, g2, g3, w0, w1, w2, w3):
    bufs = (r0, r1, r2, r3)
    gsems = (g0, g1, g2, g3)
    wsems = (w0, w1, w2, w3)

    wid = lax.axis_index("s") * _NC + lax.axis_index("c")
    base = wid * _PER_W

    pltpu.sync_copy(pos, pos_v)
    pltpu.sync_copy(idx_flat.at[pl.ds(base, _PER_W)], idxf_v)
    pltpu.sync_copy(idx2d.at[wid], idx2d_v)

    @plsc.parallel_loop(0, _PER_W // _LANES, unroll=8)
    def _(k):
        sl = pl.ds(k * _LANES, _LANES)
        mask_v[sl] = jnp.minimum(idxf_v[sl], 1)

    pltpu.sync_copy(mask_v, mask_out.at[pl.ds(base, _PER_W)])

    def issue_gather(g, buf, gsem):
        pltpu.async_copy(table.at[idx2d_v.at[2 * g]],
                         buf.at[pl.ds(0, _CHUNK)], gsem)
        pltpu.async_copy(table.at[idx2d_v.at[2 * g + 1]],
                         buf.at[pl.ds(_CHUNK, _CHUNK)], gsem)

    def wait_gather(buf, gsem):
        pltpu.make_async_copy(out.at[pl.ds(0, L)], buf, gsem).wait()

    def compute(buf):
        @plsc.parallel_loop(0, L, unroll=4)
        def _(r):
            for c in range(D // _LANES):
                sl = pl.ds(c * _LANES, _LANES)
                buf[r, sl] = buf[r, sl] * _SCALE + pos_v[r, sl]

    def issue_write(g, buf, wsem):
        pltpu.async_copy(buf, out.at[pl.ds(base + g * L, L)], wsem)

    def wait_write(buf, wsem):
        pltpu.make_async_copy(buf, out.at[pl.ds(0, L)], wsem).wait()

    issue_gather(0, bufs[0], gsems[0])
    issue_gather(1, bufs[1], gsems[1])
    for g in (0, 1):
        wait_gather(bufs[g], gsems[g])
        issue_gather(g + 2, bufs[g + 2], gsems[g + 2])
        compute(bufs[g])
        issue_write(g, bufs[g], wsems[g])

    def steady(j, carry):
        for h in range(_NB):
            g = 2 + j * _NB + h
            b = (2 + h) % _NB
            bn = (4 + h) % _NB
            wait_gather(bufs[b], gsems[b])
            wait_write(bufs[bn], wsems[bn])
            issue_gather(g + 2, bufs[bn], gsems[bn])
            compute(bufs[b])
            issue_write(g, bufs[b], wsems[b])
        return carry

    lax.fori_loop(0, (_SEQ_W - 4) // _NB, steady, 0)

    for g in (_SEQ_W - 2, _SEQ_W - 1):
        b = g % _NB
        wait_gather(bufs[b], gsems[b])
        wait_write(bufs[(b + 2) % _NB], wsems[(b + 2) % _NB])
        compute(bufs[b])
        issue_write(g, bufs[b], wsems[b])
    for g in (_SEQ_W - 2, _SEQ_W - 1):
        b = g % _NB
        wait_write(bufs[b], wsems[b])


@jax.jit
def _run(idx_flat, idx2d, pos, table):
    mesh = plsc.VectorSubcoreMesh(core_axis_name="c", subcore_axis_name="s")
    f = pl.kernel(
        _sc_body,
        out_type=(
            jax.ShapeDtypeStruct((_TOK, D), jnp.float32),
            jax.ShapeDtypeStruct((_TOK,), jnp.int32),
        ),
        mesh=mesh,
        scratch_types=(
            pltpu.VMEM((_PER_W,), jnp.int32),
            pltpu.VMEM((_SEQ_W * 2, _CHUNK), jnp.int32),
            pltpu.VMEM((L, D), jnp.float32),
            pltpu.VMEM((L, D), jnp.float32),
            pltpu.VMEM((L, D), jnp.float32),
            pltpu.VMEM((L, D), jnp.float32),
            pltpu.VMEM((L, D), jnp.float32),
            pltpu.VMEM((_PER_W,), jnp.int32),
            pltpu.SemaphoreType.DMA,
            pltpu.SemaphoreType.DMA,
            pltpu.SemaphoreType.DMA,
            pltpu.SemaphoreType.DMA,
            pltpu.SemaphoreType.DMA,
            pltpu.SemaphoreType.DMA,
            pltpu.SemaphoreType.DMA,
            pltpu.SemaphoreType.DMA,
        ),
        compiler_params=pltpu.CompilerParams(use_tc_tiling_on_sc=False),
    )
    return f(idx_flat, idx2d, pos, table)


def kernel(indices, table):
    idx_flat = indices.reshape(-1).astype(jnp.int32)
    idx2d = idx_flat.reshape(_NW, _SEQ_W * 2, _CHUNK)
    pos = _pos_enc()
    out_flat, mask_i32 = _run(idx_flat, idx2d, pos, table)
    x = out_flat.reshape(B, L, D)
    mask = mask_i32.reshape(B, L).astype(bool)
    return (x, mask)
